# baseline (speedup 1.0000x reference)
import functools

import jax
import jax.numpy as jnp
from jax import lax
from jax.experimental import pallas as pl
from jax.experimental.pallas import tpu as pltpu
from jax.experimental.pallas import tpu_sc as plsc

N_NODES = 10000
N_EDGES = 320000
D = 128
DH = 64
N_LAYERS = 5
N_GRAPHS = 16

N_PAD = 10240
ROWS_PER_TILE = N_PAD // 16
CHUNK = 128
N_SUBCORES = 16
_CPT = -(-N_EDGES // (N_SUBCORES * CHUNK))
CHUNKS_PER_TILE = -(-_CPT // 8) * 8
E_PAD = N_SUBCORES * CHUNK * CHUNKS_PER_TILE
NSETS = 2
CROWS = 4
SCHUNK = CROWS * CHUNK
SUPERCHUNKS = CHUNKS_PER_TILE // CROWS
ROW_BLK = 512
N_ROW_BLKS = N_PAD // ROW_BLK


def _sc_agg(h_stack, src2d, dst2d):
    mesh = plsc.VectorSubcoreMesh(core_axis_name="c", subcore_axis_name="s")

    @functools.partial(
        pl.kernel,
        mesh=mesh,
        compiler_params=pltpu.CompilerParams(use_tc_tiling_on_sc=False),
        out_type=jax.ShapeDtypeStruct((2, N_PAD, DH), jnp.float32),
        scratch_types=(
            [pltpu.VMEM((SCHUNK,), jnp.int32) for _ in range(NSETS)]
            + [pltpu.VMEM((SCHUNK,), jnp.int32) for _ in range(NSETS)]
            + [pltpu.VMEM((SCHUNK, DH), jnp.float32) for _ in range(NSETS)]
            + [pltpu.VMEM_SHARED((N_PAD, DH), jnp.float32)]
            + [pltpu.SemaphoreType.DMA] * (4 * NSETS)
        ),
    )
    def agg_kernel(h_hbm, src_hbm, dst_hbm, out_hbm, *sc):
        srcs = sc[0:NSETS]
        dsts = sc[NSETS:2 * NSETS]
        bufs = sc[2 * NSETS:3 * NSETS]
        acc = sc[3 * NSETS]
        gsem = sc[3 * NSETS + 1:3 * NSETS + 1 + NSETS]
        ssem = sc[3 * NSETS + 1 + NSETS:3 * NSETS + 1 + 2 * NSETS]
        isem_s = sc[3 * NSETS + 1 + 2 * NSETS:3 * NSETS + 1 + 3 * NSETS]
        isem_d = sc[3 * NSETS + 1 + 3 * NSETS:3 * NSETS + 1 + 4 * NSETS]
        c = lax.axis_index("c")
        s_ = lax.axis_index("s")
        r0 = s_ * ROWS_PER_TILE
        htab = h_hbm.at[c]
        base = s_ * CHUNKS_PER_TILE * CHUNK
        init_h = pltpu.async_copy(htab.at[pl.ds(r0, ROWS_PER_TILE)],
                                  acc.at[pl.ds(r0, ROWS_PER_TILE)], isem_d[0])

        def gath(pp, j):
            return pltpu.async_copy(htab.at[srcs[pp]], bufs[pp], gsem[pp])


        def wait_gath(pp):
            pltpu.make_async_copy(htab.at[pl.ds(0, CHUNK)], bufs[pp],
                                  gsem[pp]).wait()

        def scat(pp, j):
            return pltpu.async_copy(bufs[pp], acc.at[dsts[pp]], ssem[pp],
                                    add=True)

        def load(hbm, vbuf, sem, j):
            return pltpu.async_copy(
                hbm.at[pl.ds(base + j * SCHUNK, SCHUNK)], vbuf, sem)

        def wait_load(vbuf, sem):
            pltpu.make_async_copy(src_hbm.at[pl.ds(0, SCHUNK)], vbuf,
                                  sem).wait()

        for pp in range(NSETS):
            pltpu.sync_copy(src_hbm.at[pl.ds(base + pp * SCHUNK, SCHUNK)],
                            srcs[pp])
            pltpu.sync_copy(dst_hbm.at[pl.ds(base + pp * SCHUNK, SCHUNK)],
                            dsts[pp])
        for pp in range(NSETS):
            gath(pp, pp)
        init_h.wait()
        plsc.subcore_barrier()

        @pl.loop(0, SUPERCHUNKS, step=NSETS)
        def _(g):
            sh = [None] * NSETS
            for pp in range(NSETS):
                j = g + pp
                wait_gath(pp)

                @pl.when(j >= NSETS)
                def _(pp=pp):
                    wait_load(dsts[pp], isem_d[pp])

                sh[pp] = scat(pp, j)

                @pl.when(j + NSETS < SUPERCHUNKS)
                def _(pp=pp, j=j):
                    load(src_hbm, srcs[pp], isem_s[pp], j + NSETS)

            for pp in range(NSETS):
                j = g + pp
                sh[pp].wait()

                @pl.when(j + NSETS < SUPERCHUNKS)
                def _(pp=pp, j=j):
                    load(dst_hbm, dsts[pp], isem_d[pp], j + NSETS)
                    wait_load(srcs[pp], isem_s[pp])
                    gath(pp, j + NSETS)

        plsc.subcore_barrier()
        pltpu.sync_copy(acc.at[pl.ds(r0, ROWS_PER_TILE)],
                        out_hbm.at[c].at[pl.ds(r0, ROWS_PER_TILE)])

    return agg_kernel(h_stack, src2d, dst2d)


def _proj_kernel(x_ref, w_ref, b_ref, out_ref):
    h = jnp.dot(x_ref[...], w_ref[...], precision=lax.Precision.DEFAULT)
    h = h + b_ref[0]
    out_ref[0] = h[:, :DH]
    out_ref[1] = h[:, DH:]


def _project(x_pad, W_proj, b_proj):
    return pl.pallas_call(
        _proj_kernel,
        grid=(N_ROW_BLKS,),
        in_specs=[
            pl.BlockSpec((ROW_BLK, D), lambda r: (r, 0)),
            pl.BlockSpec((D, D), lambda r: (0, 0)),
            pl.BlockSpec((1, D), lambda r: (0, 0)),
        ],
        out_specs=pl.BlockSpec((2, ROW_BLK, DH), lambda r: (0, r, 0)),
        out_shape=jax.ShapeDtypeStruct((2, N_PAD, DH), jnp.float32),
    )(x_pad, W_proj, b_proj.reshape(1, D))


def _layer_kernel(m_ref, w1_ref, b1_ref, w2_ref, b2_ref, g_ref, bb_ref,
                  out_ref):
    mA = m_ref[0]
    mB = m_ref[1]
    t = (jnp.dot(mA, w1_ref[:DH, :], precision=lax.Precision.DEFAULT)
         + jnp.dot(mB, w1_ref[DH:, :], precision=lax.Precision.DEFAULT)
         + b1_ref[0])
    t = jnp.maximum(t, 0.0)
    u = jnp.dot(t, w2_ref[...], precision=lax.Precision.DEFAULT) + b2_ref[0]
    u = jnp.maximum(u, 0.0)
    mu = jnp.mean(u, axis=-1, keepdims=True)
    var = jnp.mean((u - mu) ** 2, axis=-1, keepdims=True)
    h = (u - mu) * lax.rsqrt(var + 1e-5) * g_ref[0] + bb_ref[0]
    out_ref[0] = h[:, :DH]
    out_ref[1] = h[:, DH:]


def _layer_mlp(m_stack, W1, b1, W2, b2, ln_g, ln_b):
    return pl.pallas_call(
        _layer_kernel,
        grid=(N_ROW_BLKS,),
        in_specs=[
            pl.BlockSpec((2, ROW_BLK, DH), lambda r: (0, r, 0)),
            pl.BlockSpec((D, D), lambda r: (0, 0)),
            pl.BlockSpec((1, D), lambda r: (0, 0)),
            pl.BlockSpec((D, D), lambda r: (0, 0)),
            pl.BlockSpec((1, D), lambda r: (0, 0)),
            pl.BlockSpec((1, D), lambda r: (0, 0)),
            pl.BlockSpec((1, D), lambda r: (0, 0)),
        ],
        out_specs=pl.BlockSpec((2, ROW_BLK, DH), lambda r: (0, r, 0)),
        out_shape=jax.ShapeDtypeStruct((2, N_PAD, DH), jnp.float32),
    )(m_stack, W1, b1.reshape(1, D), W2, b2.reshape(1, D),
      ln_g.reshape(1, D), ln_b.reshape(1, D))


def _pool_kernel(h_ref, batch_ref, wf1_ref, bf1_ref, wf2_ref, bf2_ref,
                 out_ref, g_acc):
    r = pl.program_id(0)

    @pl.when(r == 0)
    def _():
        g_acc[...] = jnp.zeros_like(g_acc)

    b = batch_ref[0, 0, :]
    gids = lax.broadcasted_iota(jnp.int32, (N_GRAPHS, ROW_BLK), 0)
    mask = (gids == b[None, :]).astype(jnp.float32)
    g_acc[:, :DH] += jnp.dot(mask, h_ref[0],
                             precision=lax.Precision.DEFAULT)
    g_acc[:, DH:] += jnp.dot(mask, h_ref[1],
                             precision=lax.Precision.DEFAULT)

    @pl.when(r == N_ROW_BLKS - 1)
    def _():
        g = g_acc[...]
        t = jnp.dot(g, wf1_ref[...], precision=lax.Precision.DEFAULT) + bf1_ref[0]
        t = jnp.maximum(t, 0.0)
        out_ref[...] = (jnp.dot(t, wf2_ref[...],
                                precision=lax.Precision.DEFAULT) + bf2_ref[0])


def _pool_mlp(h_stack, batch3d, Wf1, bf1, Wf2, bf2):
    return pl.pallas_call(
        _pool_kernel,
        grid=(N_ROW_BLKS,),
        in_specs=[
            pl.BlockSpec((2, ROW_BLK, DH), lambda r: (0, r, 0)),
            pl.BlockSpec((1, 1, ROW_BLK), lambda r: (r, 0, 0)),
            pl.BlockSpec((D, 2 * D), lambda r: (0, 0)),
            pl.BlockSpec((1, 2 * D), lambda r: (0, 0)),
            pl.BlockSpec((2 * D, D), lambda r: (0, 0)),
            pl.BlockSpec((1, D), lambda r: (0, 0)),
        ],
        out_specs=pl.BlockSpec((N_GRAPHS, D), lambda r: (0, 0)),
        out_shape=jax.ShapeDtypeStruct((N_GRAPHS, D), jnp.float32),
        scratch_shapes=[pltpu.VMEM((N_GRAPHS, D), jnp.float32)],
    )(h_stack, batch3d, Wf1, bf1.reshape(1, 2 * D), Wf2, bf2.reshape(1, D))


def kernel(x, edge_index, batch, W_proj, b_proj, W1, b1, W2, b2, ln_g, ln_b,
           Wf1, bf1, Wf2, bf2):
    x_pad = jnp.pad(x, ((0, N_PAD - N_NODES), (0, 0)))
    src = jnp.pad(edge_index[0], (0, E_PAD - N_EDGES))
    dst = jnp.pad(edge_index[1], (0, E_PAD - N_EDGES),
                  constant_values=N_PAD - 1)

    batch3d = jnp.pad(batch, (0, N_PAD - N_NODES),
                      constant_values=N_GRAPHS).reshape(N_ROW_BLKS, 1, ROW_BLK)

    h = _project(x_pad, W_proj, b_proj)
    for i in range(N_LAYERS):
        m = _sc_agg(h, src, dst)
        h = _layer_mlp(m, W1[i], b1[i], W2[i], b2[i], ln_g[i], ln_b[i])
    return _pool_mlp(h, batch3d, Wf1, bf1, Wf2, bf2)


# speedup vs baseline: 1.0341x; 1.0341x over previous
import functools

import jax
import jax.numpy as jnp
from jax import lax
from jax.experimental import pallas as pl
from jax.experimental.pallas import tpu as pltpu
from jax.experimental.pallas import tpu_sc as plsc

N_NODES = 10000
N_EDGES = 320000
D = 128
DH = 64
N_LAYERS = 5
N_GRAPHS = 16

N_PAD = 10240
ROWS_PER_TILE = N_PAD // 16
CHUNK = 128
N_SUBCORES = 16
_CPT = -(-N_EDGES // (N_SUBCORES * CHUNK))
CHUNKS_PER_TILE = -(-_CPT // 8) * 8
E_PAD = N_SUBCORES * CHUNK * CHUNKS_PER_TILE
NSETS = 5
CROWS = 2
SCHUNK = CROWS * CHUNK
SUPERCHUNKS = CHUNKS_PER_TILE // CROWS
ROW_BLK = 512
N_ROW_BLKS = N_PAD // ROW_BLK


def _sc_agg(h_stack, src2d, dst2d):
    mesh = plsc.VectorSubcoreMesh(core_axis_name="c", subcore_axis_name="s")

    @functools.partial(
        pl.kernel,
        mesh=mesh,
        compiler_params=pltpu.CompilerParams(use_tc_tiling_on_sc=False),
        out_type=jax.ShapeDtypeStruct((2, N_PAD, DH), jnp.float32),
        scratch_types=(
            [pltpu.VMEM((SCHUNK,), jnp.int32) for _ in range(NSETS)]
            + [pltpu.VMEM((SCHUNK,), jnp.int32) for _ in range(NSETS)]
            + [pltpu.VMEM((SCHUNK, DH), jnp.float32) for _ in range(NSETS)]
            + [pltpu.VMEM_SHARED((N_PAD, DH), jnp.float32)]
            + [pltpu.SemaphoreType.DMA] * (4 * NSETS)
        ),
    )
    def agg_kernel(h_hbm, src_hbm, dst_hbm, out_hbm, *sc):
        srcs = sc[0:NSETS]
        dsts = sc[NSETS:2 * NSETS]
        bufs = sc[2 * NSETS:3 * NSETS]
        acc = sc[3 * NSETS]
        gsem = sc[3 * NSETS + 1:3 * NSETS + 1 + NSETS]
        ssem = sc[3 * NSETS + 1 + NSETS:3 * NSETS + 1 + 2 * NSETS]
        isem_s = sc[3 * NSETS + 1 + 2 * NSETS:3 * NSETS + 1 + 3 * NSETS]
        isem_d = sc[3 * NSETS + 1 + 3 * NSETS:3 * NSETS + 1 + 4 * NSETS]
        c = lax.axis_index("c")
        s_ = lax.axis_index("s")
        r0 = s_ * ROWS_PER_TILE
        htab = h_hbm.at[c]
        base = s_ * CHUNKS_PER_TILE * CHUNK
        init_h = pltpu.async_copy(htab.at[pl.ds(r0, ROWS_PER_TILE)],
                                  acc.at[pl.ds(r0, ROWS_PER_TILE)], isem_d[0])

        def gath(pp, j):
            return pltpu.async_copy(htab.at[srcs[pp]], bufs[pp], gsem[pp])


        def wait_gath(pp):
            pltpu.make_async_copy(htab.at[pl.ds(0, CHUNK)], bufs[pp],
                                  gsem[pp]).wait()

        def scat(pp, j):
            return pltpu.async_copy(bufs[pp], acc.at[dsts[pp]], ssem[pp],
                                    add=True)

        def load(hbm, vbuf, sem, j):
            return pltpu.async_copy(
                hbm.at[pl.ds(base + j * SCHUNK, SCHUNK)], vbuf, sem)

        def wait_load(vbuf, sem):
            pltpu.make_async_copy(src_hbm.at[pl.ds(0, SCHUNK)], vbuf,
                                  sem).wait()

        for pp in range(NSETS):
            pltpu.sync_copy(src_hbm.at[pl.ds(base + pp * SCHUNK, SCHUNK)],
                            srcs[pp])
            pltpu.sync_copy(dst_hbm.at[pl.ds(base + pp * SCHUNK, SCHUNK)],
                            dsts[pp])
        for pp in range(NSETS):
            gath(pp, pp)
        init_h.wait()
        plsc.subcore_barrier()

        @pl.loop(0, SUPERCHUNKS, step=NSETS)
        def _(g):
            sh = [None] * NSETS
            for pp in range(NSETS):
                j = g + pp
                wait_gath(pp)

                @pl.when(j >= NSETS)
                def _(pp=pp):
                    wait_load(dsts[pp], isem_d[pp])

                sh[pp] = scat(pp, j)

                @pl.when(j + NSETS < SUPERCHUNKS)
                def _(pp=pp, j=j):
                    load(src_hbm, srcs[pp], isem_s[pp], j + NSETS)

            for pp in range(NSETS):
                j = g + pp
                sh[pp].wait()

                @pl.when(j + NSETS < SUPERCHUNKS)
                def _(pp=pp, j=j):
                    load(dst_hbm, dsts[pp], isem_d[pp], j + NSETS)
                    wait_load(srcs[pp], isem_s[pp])
                    gath(pp, j + NSETS)

        plsc.subcore_barrier()
        pltpu.sync_copy(acc.at[pl.ds(r0, ROWS_PER_TILE)],
                        out_hbm.at[c].at[pl.ds(r0, ROWS_PER_TILE)])

    return agg_kernel(h_stack, src2d, dst2d)


def _proj_kernel(x_ref, w_ref, b_ref, out_ref):
    h = jnp.dot(x_ref[...], w_ref[...], precision=lax.Precision.DEFAULT)
    h = h + b_ref[0]
    out_ref[0] = h[:, :DH]
    out_ref[1] = h[:, DH:]


def _project(x_pad, W_proj, b_proj):
    return pl.pallas_call(
        _proj_kernel,
        grid=(N_ROW_BLKS,),
        in_specs=[
            pl.BlockSpec((ROW_BLK, D), lambda r: (r, 0)),
            pl.BlockSpec((D, D), lambda r: (0, 0)),
            pl.BlockSpec((1, D), lambda r: (0, 0)),
        ],
        out_specs=pl.BlockSpec((2, ROW_BLK, DH), lambda r: (0, r, 0)),
        out_shape=jax.ShapeDtypeStruct((2, N_PAD, DH), jnp.float32),
    )(x_pad, W_proj, b_proj.reshape(1, D))


def _layer_kernel(m_ref, w1_ref, b1_ref, w2_ref, b2_ref, g_ref, bb_ref,
                  out_ref):
    mA = m_ref[0]
    mB = m_ref[1]
    t = (jnp.dot(mA, w1_ref[:DH, :], precision=lax.Precision.DEFAULT)
         + jnp.dot(mB, w1_ref[DH:, :], precision=lax.Precision.DEFAULT)
         + b1_ref[0])
    t = jnp.maximum(t, 0.0)
    u = jnp.dot(t, w2_ref[...], precision=lax.Precision.DEFAULT) + b2_ref[0]
    u = jnp.maximum(u, 0.0)
    mu = jnp.mean(u, axis=-1, keepdims=True)
    var = jnp.mean((u - mu) ** 2, axis=-1, keepdims=True)
    h = (u - mu) * lax.rsqrt(var + 1e-5) * g_ref[0] + bb_ref[0]
    out_ref[0] = h[:, :DH]
    out_ref[1] = h[:, DH:]


def _layer_mlp(m_stack, W1, b1, W2, b2, ln_g, ln_b):
    return pl.pallas_call(
        _layer_kernel,
        grid=(N_ROW_BLKS,),
        in_specs=[
            pl.BlockSpec((2, ROW_BLK, DH), lambda r: (0, r, 0)),
            pl.BlockSpec((D, D), lambda r: (0, 0)),
            pl.BlockSpec((1, D), lambda r: (0, 0)),
            pl.BlockSpec((D, D), lambda r: (0, 0)),
            pl.BlockSpec((1, D), lambda r: (0, 0)),
            pl.BlockSpec((1, D), lambda r: (0, 0)),
            pl.BlockSpec((1, D), lambda r: (0, 0)),
        ],
        out_specs=pl.BlockSpec((2, ROW_BLK, DH), lambda r: (0, r, 0)),
        out_shape=jax.ShapeDtypeStruct((2, N_PAD, DH), jnp.float32),
    )(m_stack, W1, b1.reshape(1, D), W2, b2.reshape(1, D),
      ln_g.reshape(1, D), ln_b.reshape(1, D))


def _pool_kernel(h_ref, batch_ref, wf1_ref, bf1_ref, wf2_ref, bf2_ref,
                 out_ref, g_acc):
    r = pl.program_id(0)

    @pl.when(r == 0)
    def _():
        g_acc[...] = jnp.zeros_like(g_acc)

    b = batch_ref[0, 0, :]
    gids = lax.broadcasted_iota(jnp.int32, (N_GRAPHS, ROW_BLK), 0)
    mask = (gids == b[None, :]).astype(jnp.float32)
    g_acc[:, :DH] += jnp.dot(mask, h_ref[0],
                             precision=lax.Precision.DEFAULT)
    g_acc[:, DH:] += jnp.dot(mask, h_ref[1],
                             precision=lax.Precision.DEFAULT)

    @pl.when(r == N_ROW_BLKS - 1)
    def _():
        g = g_acc[...]
        t = jnp.dot(g, wf1_ref[...], precision=lax.Precision.DEFAULT) + bf1_ref[0]
        t = jnp.maximum(t, 0.0)
        out_ref[...] = (jnp.dot(t, wf2_ref[...],
                                precision=lax.Precision.DEFAULT) + bf2_ref[0])


def _pool_mlp(h_stack, batch3d, Wf1, bf1, Wf2, bf2):
    return pl.pallas_call(
        _pool_kernel,
        grid=(N_ROW_BLKS,),
        in_specs=[
            pl.BlockSpec((2, ROW_BLK, DH), lambda r: (0, r, 0)),
            pl.BlockSpec((1, 1, ROW_BLK), lambda r: (r, 0, 0)),
            pl.BlockSpec((D, 2 * D), lambda r: (0, 0)),
            pl.BlockSpec((1, 2 * D), lambda r: (0, 0)),
            pl.BlockSpec((2 * D, D), lambda r: (0, 0)),
            pl.BlockSpec((1, D), lambda r: (0, 0)),
        ],
        out_specs=pl.BlockSpec((N_GRAPHS, D), lambda r: (0, 0)),
        out_shape=jax.ShapeDtypeStruct((N_GRAPHS, D), jnp.float32),
        scratch_shapes=[pltpu.VMEM((N_GRAPHS, D), jnp.float32)],
    )(h_stack, batch3d, Wf1, bf1.reshape(1, 2 * D), Wf2, bf2.reshape(1, D))


def kernel(x, edge_index, batch, W_proj, b_proj, W1, b1, W2, b2, ln_g, ln_b,
           Wf1, bf1, Wf2, bf2):
    x_pad = jnp.pad(x, ((0, N_PAD - N_NODES), (0, 0)))
    src = jnp.pad(edge_index[0], (0, E_PAD - N_EDGES))
    dst = jnp.pad(edge_index[1], (0, E_PAD - N_EDGES),
                  constant_values=N_PAD - 1)

    batch3d = jnp.pad(batch, (0, N_PAD - N_NODES),
                      constant_values=N_GRAPHS).reshape(N_ROW_BLKS, 1, ROW_BLK)

    h = _project(x_pad, W_proj, b_proj)
    for i in range(N_LAYERS):
        m = _sc_agg(h, src, dst)
        h = _layer_mlp(m, W1[i], b1[i], W2[i], b2[i], ln_g[i], ln_b[i])
    return _pool_mlp(h, batch3d, Wf1, bf1, Wf2, bf2)


# speedup vs baseline: 1.0462x; 1.0117x over previous
import functools

import jax
import jax.numpy as jnp
from jax import lax
from jax.experimental import pallas as pl
from jax.experimental.pallas import tpu as pltpu
from jax.experimental.pallas import tpu_sc as plsc

N_NODES = 10000
N_EDGES = 320000
D = 128
DH = 64
N_LAYERS = 5
N_GRAPHS = 16

N_PAD = 10240
ROWS_PER_TILE = N_PAD // 16
CHUNK = 128
N_SUBCORES = 16
_CPT = -(-N_EDGES // (N_SUBCORES * CHUNK))
CHUNKS_PER_TILE = -(-_CPT // 8) * 8
E_PAD = N_SUBCORES * CHUNK * CHUNKS_PER_TILE
NSETS = 4
CROWS = 2
SCHUNK = CROWS * CHUNK
SUPERCHUNKS = CHUNKS_PER_TILE // CROWS
ROW_BLK = 512
N_ROW_BLKS = N_PAD // ROW_BLK


def _sc_agg(h_stack, src2d, dst2d):
    mesh = plsc.VectorSubcoreMesh(core_axis_name="c", subcore_axis_name="s")

    @functools.partial(
        pl.kernel,
        mesh=mesh,
        compiler_params=pltpu.CompilerParams(use_tc_tiling_on_sc=False),
        out_type=jax.ShapeDtypeStruct((2, N_PAD, DH), jnp.float32),
        scratch_types=(
            [pltpu.VMEM((SCHUNK,), jnp.int32) for _ in range(NSETS)]
            + [pltpu.VMEM((SCHUNK,), jnp.int32) for _ in range(NSETS)]
            + [pltpu.VMEM((SCHUNK, DH), jnp.float32) for _ in range(NSETS)]
            + [pltpu.VMEM_SHARED((N_PAD, DH), jnp.float32)]
            + [pltpu.SemaphoreType.DMA] * (4 * NSETS)
        ),
    )
    def agg_kernel(h_hbm, src_hbm, dst_hbm, out_hbm, *sc):
        srcs = sc[0:NSETS]
        dsts = sc[NSETS:2 * NSETS]
        bufs = sc[2 * NSETS:3 * NSETS]
        acc = sc[3 * NSETS]
        gsem = sc[3 * NSETS + 1:3 * NSETS + 1 + NSETS]
        ssem = sc[3 * NSETS + 1 + NSETS:3 * NSETS + 1 + 2 * NSETS]
        isem_s = sc[3 * NSETS + 1 + 2 * NSETS:3 * NSETS + 1 + 3 * NSETS]
        isem_d = sc[3 * NSETS + 1 + 3 * NSETS:3 * NSETS + 1 + 4 * NSETS]
        c = lax.axis_index("c")
        s_ = lax.axis_index("s")
        r0 = s_ * ROWS_PER_TILE
        htab = h_hbm.at[c]
        base = s_ * CHUNKS_PER_TILE * CHUNK
        init_h = pltpu.async_copy(htab.at[pl.ds(r0, ROWS_PER_TILE)],
                                  acc.at[pl.ds(r0, ROWS_PER_TILE)], isem_d[0])

        def gath(pp, j):
            return pltpu.async_copy(htab.at[srcs[pp]], bufs[pp], gsem[pp])


        def wait_gath(pp):
            pltpu.make_async_copy(htab.at[pl.ds(0, CHUNK)], bufs[pp],
                                  gsem[pp]).wait()

        def scat(pp, j):
            return pltpu.async_copy(bufs[pp], acc.at[dsts[pp]], ssem[pp],
                                    add=True)

        def load(hbm, vbuf, sem, j):
            return pltpu.async_copy(
                hbm.at[pl.ds(base + j * SCHUNK, SCHUNK)], vbuf, sem)

        def wait_load(vbuf, sem):
            pltpu.make_async_copy(src_hbm.at[pl.ds(0, SCHUNK)], vbuf,
                                  sem).wait()

        for pp in range(NSETS):
            pltpu.sync_copy(src_hbm.at[pl.ds(base + pp * SCHUNK, SCHUNK)],
                            srcs[pp])
            pltpu.sync_copy(dst_hbm.at[pl.ds(base + pp * SCHUNK, SCHUNK)],
                            dsts[pp])
        for pp in range(NSETS):
            gath(pp, pp)
        init_h.wait()
        plsc.subcore_barrier()

        @pl.loop(0, SUPERCHUNKS, step=NSETS)
        def _(g):
            sh = [None] * NSETS
            for pp in range(NSETS):
                j = g + pp
                wait_gath(pp)

                @pl.when(j >= NSETS)
                def _(pp=pp):
                    wait_load(dsts[pp], isem_d[pp])

                sh[pp] = scat(pp, j)

                @pl.when(j + NSETS < SUPERCHUNKS)
                def _(pp=pp, j=j):
                    load(src_hbm, srcs[pp], isem_s[pp], j + NSETS)

            for pp in range(NSETS):
                j = g + pp
                sh[pp].wait()

                @pl.when(j + NSETS < SUPERCHUNKS)
                def _(pp=pp, j=j):
                    load(dst_hbm, dsts[pp], isem_d[pp], j + NSETS)
                    wait_load(srcs[pp], isem_s[pp])
                    gath(pp, j + NSETS)

        plsc.subcore_barrier()
        pltpu.sync_copy(acc.at[pl.ds(r0, ROWS_PER_TILE)],
                        out_hbm.at[c].at[pl.ds(r0, ROWS_PER_TILE)])

    return agg_kernel(h_stack, src2d, dst2d)


def _proj_kernel(x_ref, w_ref, b_ref, out_ref):
    h = jnp.dot(x_ref[...], w_ref[...], precision=lax.Precision.DEFAULT)
    h = h + b_ref[0]
    out_ref[0] = h[:, :DH]
    out_ref[1] = h[:, DH:]


def _project(x_pad, W_proj, b_proj):
    return pl.pallas_call(
        _proj_kernel,
        grid=(N_ROW_BLKS,),
        in_specs=[
            pl.BlockSpec((ROW_BLK, D), lambda r: (r, 0)),
            pl.BlockSpec((D, D), lambda r: (0, 0)),
            pl.BlockSpec((1, D), lambda r: (0, 0)),
        ],
        out_specs=pl.BlockSpec((2, ROW_BLK, DH), lambda r: (0, r, 0)),
        out_shape=jax.ShapeDtypeStruct((2, N_PAD, DH), jnp.float32),
    )(x_pad, W_proj, b_proj.reshape(1, D))


def _layer_kernel(m_ref, w1_ref, b1_ref, w2_ref, b2_ref, g_ref, bb_ref,
                  out_ref):
    mA = m_ref[0]
    mB = m_ref[1]
    t = (jnp.dot(mA, w1_ref[:DH, :], precision=lax.Precision.DEFAULT)
         + jnp.dot(mB, w1_ref[DH:, :], precision=lax.Precision.DEFAULT)
         + b1_ref[0])
    t = jnp.maximum(t, 0.0)
    u = jnp.dot(t, w2_ref[...], precision=lax.Precision.DEFAULT) + b2_ref[0]
    u = jnp.maximum(u, 0.0)
    mu = jnp.mean(u, axis=-1, keepdims=True)
    var = jnp.mean((u - mu) ** 2, axis=-1, keepdims=True)
    h = (u - mu) * lax.rsqrt(var + 1e-5) * g_ref[0] + bb_ref[0]
    out_ref[0] = h[:, :DH]
    out_ref[1] = h[:, DH:]


def _layer_mlp(m_stack, W1, b1, W2, b2, ln_g, ln_b):
    return pl.pallas_call(
        _layer_kernel,
        grid=(N_ROW_BLKS,),
        in_specs=[
            pl.BlockSpec((2, ROW_BLK, DH), lambda r: (0, r, 0)),
            pl.BlockSpec((D, D), lambda r: (0, 0)),
            pl.BlockSpec((1, D), lambda r: (0, 0)),
            pl.BlockSpec((D, D), lambda r: (0, 0)),
            pl.BlockSpec((1, D), lambda r: (0, 0)),
            pl.BlockSpec((1, D), lambda r: (0, 0)),
            pl.BlockSpec((1, D), lambda r: (0, 0)),
        ],
        out_specs=pl.BlockSpec((2, ROW_BLK, DH), lambda r: (0, r, 0)),
        out_shape=jax.ShapeDtypeStruct((2, N_PAD, DH), jnp.float32),
    )(m_stack, W1, b1.reshape(1, D), W2, b2.reshape(1, D),
      ln_g.reshape(1, D), ln_b.reshape(1, D))


def _pool_kernel(m_ref, w1_ref, b1_ref, w2_ref, b2_ref, g_ref, bb_ref,
                 batch_ref, wf1_ref, bf1_ref, wf2_ref, bf2_ref,
                 out_ref, g_acc):
    r = pl.program_id(0)

    @pl.when(r == 0)
    def _():
        g_acc[...] = jnp.zeros_like(g_acc)

    mA = m_ref[0]
    mB = m_ref[1]
    t = (jnp.dot(mA, w1_ref[:DH, :], precision=lax.Precision.DEFAULT)
         + jnp.dot(mB, w1_ref[DH:, :], precision=lax.Precision.DEFAULT)
         + b1_ref[0])
    t = jnp.maximum(t, 0.0)
    u = jnp.dot(t, w2_ref[...], precision=lax.Precision.DEFAULT) + b2_ref[0]
    u = jnp.maximum(u, 0.0)
    mu = jnp.mean(u, axis=-1, keepdims=True)
    var = jnp.mean((u - mu) ** 2, axis=-1, keepdims=True)
    h = (u - mu) * lax.rsqrt(var + 1e-5) * g_ref[0] + bb_ref[0]

    b = batch_ref[0, 0, :]
    gids = lax.broadcasted_iota(jnp.int32, (N_GRAPHS, ROW_BLK), 0)
    mask = (gids == b[None, :]).astype(jnp.float32)
    g_acc[...] += jnp.dot(mask, h, precision=lax.Precision.DEFAULT)

    @pl.when(r == N_ROW_BLKS - 1)
    def _():
        g = g_acc[...]
        tt = jnp.dot(g, wf1_ref[...], precision=lax.Precision.DEFAULT) + bf1_ref[0]
        tt = jnp.maximum(tt, 0.0)
        out_ref[...] = (jnp.dot(tt, wf2_ref[...],
                                precision=lax.Precision.DEFAULT) + bf2_ref[0])


def _pool_mlp(m_stack, W1, b1, W2, b2, ln_g, ln_b, batch3d, Wf1, bf1, Wf2,
              bf2):
    full = lambda shape: pl.BlockSpec(shape, lambda r: tuple(0 for _ in shape))
    return pl.pallas_call(
        _pool_kernel,
        grid=(N_ROW_BLKS,),
        in_specs=[
            pl.BlockSpec((2, ROW_BLK, DH), lambda r: (0, r, 0)),
            full((D, D)),
            full((1, D)),
            full((D, D)),
            full((1, D)),
            full((1, D)),
            full((1, D)),
            pl.BlockSpec((1, 1, ROW_BLK), lambda r: (r, 0, 0)),
            full((D, 2 * D)),
            full((1, 2 * D)),
            full((2 * D, D)),
            full((1, D)),
        ],
        out_specs=pl.BlockSpec((N_GRAPHS, D), lambda r: (0, 0)),
        out_shape=jax.ShapeDtypeStruct((N_GRAPHS, D), jnp.float32),
        scratch_shapes=[pltpu.VMEM((N_GRAPHS, D), jnp.float32)],
    )(m_stack, W1, b1.reshape(1, D), W2, b2.reshape(1, D),
      ln_g.reshape(1, D), ln_b.reshape(1, D), batch3d,
      Wf1, bf1.reshape(1, 2 * D), Wf2, bf2.reshape(1, D))


def kernel(x, edge_index, batch, W_proj, b_proj, W1, b1, W2, b2, ln_g, ln_b,
           Wf1, bf1, Wf2, bf2):
    x_pad = jnp.pad(x, ((0, N_PAD - N_NODES), (0, 0)))
    src = jnp.pad(edge_index[0], (0, E_PAD - N_EDGES))
    dst = jnp.pad(edge_index[1], (0, E_PAD - N_EDGES),
                  constant_values=N_PAD - 1)

    batch3d = jnp.pad(batch, (0, N_PAD - N_NODES),
                      constant_values=N_GRAPHS).reshape(N_ROW_BLKS, 1, ROW_BLK)

    h = _project(x_pad, W_proj, b_proj)
    for i in range(N_LAYERS - 1):
        m = _sc_agg(h, src, dst)
        h = _layer_mlp(m, W1[i], b1[i], W2[i], b2[i], ln_g[i], ln_b[i])
    m = _sc_agg(h, src, dst)
    i = N_LAYERS - 1
    return _pool_mlp(m, W1[i], b1[i], W2[i], b2[i], ln_g[i], ln_b[i],
                     batch3d, Wf1, bf1, Wf2, bf2)
